# Initial kernel scaffold; baseline (speedup 1.0000x reference)
import jax
import jax.numpy as jnp
from jax.experimental import pallas as pl


def kernel(hv, hc, vadj_row, vadj_col, vadj_val, cadj_row, cadj_col, cadj_val, params):
    raise NotImplementedError("write your pallas kernel here")


# speedup vs baseline: 4.6549x; 4.6549x over previous
import functools

import jax
import jax.numpy as jnp
from jax import lax
from jax.experimental import pallas as pl
from jax.experimental.pallas import tpu as pltpu
from jax.experimental.pallas import tpu_sc as plsc

_N = 10000
_E = 320000
_DH = 256
_MB = 1000
_NSUB = 16
_EC = _E // _NSUB
_CK = 80
_NG = 5
_GCH = 50


def _dot(a, b):
    return jnp.dot(a, b, preferred_element_type=jnp.float32)


def _msg_body(x_ref, w1_ref, b1_ref, w2_ref, b2_ref, o_ref):
    h = jnp.maximum(_dot(x_ref[...], w1_ref[0]) + b1_ref[0], 0.0)
    y = jnp.maximum(_dot(h, w2_ref[0]) + b2_ref[0], 0.0)
    o_ref[0] = y[:, :128]
    o_ref[1] = y[:, 128:]


def _msg(x, w1s, b1s, w2s, b2s):
    n, din = x.shape
    nb = n // _MB
    return pl.pallas_call(
        _msg_body,
        grid=(2, nb),
        in_specs=[
            pl.BlockSpec((_MB, din), lambda p, i: (i, 0)),
            pl.BlockSpec((1, din, _DH), lambda p, i: (p, 0, 0)),
            pl.BlockSpec((1, 1, _DH), lambda p, i: (p, 0, 0)),
            pl.BlockSpec((1, _DH, _DH), lambda p, i: (p, 0, 0)),
            pl.BlockSpec((1, 1, _DH), lambda p, i: (p, 0, 0)),
        ],
        out_specs=pl.BlockSpec((2, _MB, 128), lambda p, i: (0, p * nb + i, 0)),
        out_shape=jax.ShapeDtypeStruct((2, 2 * n, 128), jnp.float32),
    )(x, w1s, b1s, w2s, b2s)


def _lin_relu_body(x_ref, w_ref, b_ref, o_ref):
    o_ref[...] = jnp.maximum(_dot(x_ref[...], w_ref[...]) + b_ref[...], 0.0)


def _lin_relu(x, w, b):
    n, k = x.shape
    d = w.shape[1]
    nb = n // _MB
    return pl.pallas_call(
        _lin_relu_body,
        grid=(nb,),
        in_specs=[
            pl.BlockSpec((_MB, k), lambda i: (i, 0)),
            pl.BlockSpec((k, d), lambda i: (0, 0)),
            pl.BlockSpec((1, d), lambda i: (0, 0)),
        ],
        out_specs=pl.BlockSpec((_MB, d), lambda i: (i, 0)),
        out_shape=jax.ShapeDtypeStruct((n, d), jnp.float32),
    )(x, w, b.reshape(1, d))


_BNP = 10240


def _fold_sum(scr):
    w = _BNP
    while w > 5:
        h = w // 2
        scr[pl.ds(0, h), :] = scr[pl.ds(0, h), :] + scr[pl.ds(h, h), :]
        w = h
    s = scr[pl.ds(0, 1), :]
    for j in range(1, 5):
        s = s + scr[pl.ds(j, 1), :]
    return s


def _bn_body(x_ref, g_ref, b_ref, o_ref, scr):
    x = x_ref[...]
    n = x.shape[0]
    inv_n = jnp.float32(1.0 / n)
    scr[pl.ds(0, n), :] = x
    scr[pl.ds(n, _BNP - n), :] = jnp.zeros((_BNP - n, x.shape[1]), jnp.float32)
    mu = _fold_sum(scr) * inv_n
    xc = x - mu
    scr[pl.ds(0, n), :] = xc * xc
    scr[pl.ds(n, _BNP - n), :] = jnp.zeros((_BNP - n, x.shape[1]), jnp.float32)
    var = _fold_sum(scr) * inv_n
    o_ref[...] = g_ref[...] * xc * lax.rsqrt(var + 1e-5) + b_ref[...]


def _bn(x, g, b):
    n, d = x.shape
    return pl.pallas_call(
        _bn_body,
        out_shape=jax.ShapeDtypeStruct((n, d), jnp.float32),
        scratch_shapes=[pltpu.VMEM((_BNP, d), jnp.float32)],
    )(x, g.reshape(1, d), b.reshape(1, d))


def _spmm(srcT, row4, col4, val4):
    src_rows = srcT.shape[0] // 2
    mesh = plsc.VectorSubcoreMesh(core_axis_name="c", subcore_axis_name="s")

    @functools.partial(
        pl.kernel,
        out_type=jax.ShapeDtypeStruct((2 * _N, 128), jnp.float32),
        mesh=mesh,
        scratch_types=[
            pltpu.VMEM((_GCH, _CK), jnp.int32),
            pltpu.VMEM((_GCH, _CK), jnp.int32),
            pltpu.VMEM((_GCH, _CK), jnp.float32),
            pltpu.VMEM((2, _CK, 128), jnp.float32),
            pltpu.VMEM((32, 128), jnp.float32),
            pltpu.VMEM_SHARED((_N, 128), jnp.float32),
            pltpu.SemaphoreType.DMA,
            pltpu.SemaphoreType.DMA,
        ],
    )
    def k(src_hbm, row_hbm, col_hbm, val_hbm, out_hbm,
          rows_v, cols_v, vals_v, gath_v, zero_v, acc_sh, sem0, sem1):
        c = lax.axis_index("c")
        s = lax.axis_index("s")
        off = c * src_rows
        sems = (sem0, sem1)

        def zfill(t, carry):
            i = t // 8
            j = t % 8
            zero_v[i, pl.ds(j * 16, 16)] = jnp.zeros((16,), jnp.float32)
            return carry

        lax.fori_loop(0, 32 * 8, zfill, 0)
        for z in range(20):
            pltpu.sync_copy(zero_v, acc_sh.at[pl.ds(s * 624 + z * 32, 32)])
        plsc.subcore_barrier()

        for g in range(_NG):
            pltpu.sync_copy(row_hbm.at[s, g], rows_v)
            pltpu.sync_copy(col_hbm.at[s, g], cols_v)
            pltpu.sync_copy(val_hbm.at[s, g], vals_v)

            def adj(t, carry):
                i = t // (_CK // 16)
                j = t % (_CK // 16)
                sl = pl.ds(j * 16, 16)
                cols_v[i, sl] = cols_v[i, sl] + off
                return carry

            lax.fori_loop(0, _GCH * (_CK // 16), adj, 0)

            pltpu.async_copy(src_hbm.at[cols_v.at[0]], gath_v.at[0], sem0)

            def body(t, carry):
                for b in range(2):
                    i = 2 * t + b
                    pltpu.make_async_copy(
                        src_hbm.at[cols_v.at[i]], gath_v.at[b], sems[b]).wait()
                    if b == 0:
                        pltpu.async_copy(
                            src_hbm.at[cols_v.at[i + 1]], gath_v.at[1], sems[1])
                    else:
                        @pl.when(t < _GCH // 2 - 1)
                        def _():
                            pltpu.async_copy(
                                src_hbm.at[cols_v.at[i + 1]], gath_v.at[0],
                                sems[0])

                    def scale(q, carry2):
                        val16 = vals_v[i, pl.ds(q * 16, 16)]
                        e0 = q * 16
                        for kk in range(16):
                            v = val16[kk]
                            for j in range(8):
                                sl = pl.ds(j * 16, 16)
                                gath_v[b, e0 + kk, sl] = (
                                    gath_v[b, e0 + kk, sl] * v)
                        return carry2

                    lax.fori_loop(0, _CK // 16, scale, 0)
                    pltpu.sync_copy(gath_v.at[b], acc_sh.at[rows_v.at[i]],
                                    add=True)
                return carry

            lax.fori_loop(0, _GCH // 2, body, 0)

        plsc.subcore_barrier()
        pltpu.sync_copy(acc_sh.at[pl.ds(s * 624, 640)],
                        out_hbm.at[pl.ds(c * _N + s * 624, 640)])

    return k(srcT, row4, col4, val4)


def _stack_msg(pa, pb):
    return (jnp.stack([pa["l1"]["w"], pb["l1"]["w"]]),
            jnp.stack([pa["l1"]["b"], pb["l1"]["b"]])[:, None, :],
            jnp.stack([pa["l2"]["w"], pb["l2"]["w"]]),
            jnp.stack([pa["l2"]["b"], pb["l2"]["b"]])[:, None, :])


def _sorted_edges(row, col, val):
    perm = jnp.argsort(row, stable=True)
    shp = (_NSUB, _NG, _GCH, _CK)
    return (row[perm].reshape(shp), col[perm].reshape(shp),
            val[perm].reshape(shp))


def _fu(x, m, p):
    xm = jnp.concatenate([x, m[:_N], m[_N:]], axis=1)
    h = _lin_relu(xm, p["l1"]["w"], p["l1"]["b"])
    return _lin_relu(h, p["l2"]["w"], p["l2"]["b"])


def kernel(hv, hc, vadj_row, vadj_col, vadj_val, cadj_row, cadj_col, cadj_val,
           params):
    vr3, vc3, vv3 = _sorted_edges(vadj_row, vadj_col, vadj_val)
    cr3, cc3, cv3 = _sorted_edges(cadj_row, cadj_col, cadj_val)

    for i in range(5):
        p = params["convs"][i]
        mvT = _msg(hc, *_stack_msg(p["fmv_pos"], p["fmv_neg"]))
        mcT = _msg(hv, *_stack_msg(p["fmc_pos"], p["fmc_neg"]))
        mv = _spmm(mvT.reshape(4 * _N, 128), vr3, vc3, vv3)
        mc = _spmm(mcT.reshape(4 * _N, 128), cr3, cc3, cv3)
        hv = _fu(hv, mv, p["fuv"])
        hc = _fu(hc, mc, p["fuc"])
        bn = params["bns"][i]
        hv = _bn(hv, bn["g0"], bn["b0"])
        hc = _bn(hc, bn["g1"], bn["b1"])
    return (hv, hc)
